# Initial kernel scaffold; baseline (speedup 1.0000x reference)
import jax
import jax.numpy as jnp
from jax.experimental import pallas as pl


def kernel(x, edge_index, batch, global_features, params):
    raise NotImplementedError("write your pallas kernel here")


# speedup vs baseline: 5.4776x; 5.4776x over previous
import math

import jax
import jax.numpy as jnp
from jax import lax
from jax.experimental import pallas as pl
from jax.experimental.pallas import tpu as pltpu
from jax.experimental.pallas import tpu_sc as plsc

N = 50000
E = 800000
G = 128
HID = 64
HALF = 25000
HPAD = 25088
RPT = HPAD // 16
EPT = E // 16
CH = 128
NFULL = EPT // CH
TAIL = EPT - NFULL * CH

_INV_SQRT2 = 1.0 / math.sqrt(2.0)


def _gelu(f):
    return 0.5 * f * (1.0 + lax.erf(f * _INV_SQRT2))


def _layernorm(f, g, b):
    mu = jnp.mean(f, axis=-1, keepdims=True)
    var = jnp.mean((f - mu) ** 2, axis=-1, keepdims=True)
    return (f - mu) / jnp.sqrt(var + 1e-5) * g + b


def _sc_segsum(u, src, dst):
    mesh = plsc.VectorSubcoreMesh(core_axis_name="c", subcore_axis_name="s")

    def body(u_hbm, src_hbm, dst_hbm, out_hbm, acc, srcb, dstb, dstl, rows, sems):
        c = lax.axis_index("c")
        s = lax.axis_index("s")
        base = c * HALF
        trash = HALF + s

        def zr(r, _):
            for k in range(4):
                rows[0, r, pl.ds(k * 16, 16)] = jnp.zeros((16,), jnp.float32)
            return 0

        lax.fori_loop(0, CH, zr, 0)
        row0 = s * RPT
        for j in range(12):
            pltpu.sync_copy(rows.at[0], acc.at[pl.ds(row0 + j * CH, CH)])
        pltpu.sync_copy(rows.at[0, pl.ds(0, 32)], acc.at[pl.ds(row0 + 12 * CH, 32)])
        plsc.subcore_barrier()

        def load_idx(i, b):
            off = s * EPT + i * CH
            pltpu.sync_copy(src_hbm.at[pl.ds(off, CH)], srcb.at[b, 0])
            pltpu.sync_copy(dst_hbm.at[pl.ds(off, CH)], dstb.at[b, 0])

        def start_gather(b):
            pltpu.make_async_copy(u_hbm.at[srcb.at[b, 0]], rows.at[b], sems.at[b]).start()

        def wait_gather(b):
            pltpu.make_async_copy(u_hbm.at[srcb.at[b, 0]], rows.at[b], sems.at[b]).wait()

        def compute_dstl(b):
            for j in range(8):
                d = dstb[b, 0, pl.ds(j * 16, 16)]
                dl = d - base
                ok = (dl >= 0) & (dl < HALF)
                dstl[b, 0, pl.ds(j * 16, 16)] = jnp.where(ok, dl, trash)

        def scatter(b):
            pltpu.sync_copy(rows.at[b], acc.at[dstl.at[b, 0]], add=True)

        load_idx(0, 0)
        start_gather(0)

        def ring(k, _):
            for b in range(2):
                i = 2 * k + b

                @pl.when(i + 1 < NFULL)
                def _():
                    load_idx(i + 1, 1 - b)
                    start_gather(1 - b)

                compute_dstl(b)
                wait_gather(b)
                scatter(b)
            return 0

        lax.fori_loop(0, NFULL // 2, ring, 0)

        off = s * EPT + NFULL * CH
        pltpu.sync_copy(src_hbm.at[pl.ds(off, TAIL)], srcb.at[0, 0, pl.ds(0, TAIL)])
        pltpu.sync_copy(dst_hbm.at[pl.ds(off, TAIL)], dstb.at[0, 0, pl.ds(0, TAIL)])
        for j in range(TAIL // 16):
            d = dstb[0, 0, pl.ds(j * 16, 16)]
            dl = d - base
            ok = (dl >= 0) & (dl < HALF)
            dstl[0, 0, pl.ds(j * 16, 16)] = jnp.where(ok, dl, trash)
        for j in range(TAIL // 16, 8):
            dstl[0, 0, pl.ds(j * 16, 16)] = jnp.full((16,), trash, jnp.int32)
        pltpu.make_async_copy(u_hbm.at[srcb.at[0, 0]], rows.at[0], sems.at[0]).start()
        wait_gather(0)
        scatter(0)

        plsc.subcore_barrier()
        pltpu.sync_copy(acc.at[pl.ds(row0, RPT)], out_hbm.at[c, pl.ds(row0, RPT)])

    f = pl.kernel(
        body,
        out_type=jax.ShapeDtypeStruct((2, HPAD, HID), jnp.float32),
        mesh=mesh,
        scratch_types=[
            pltpu.VMEM_SHARED((HPAD, HID), jnp.float32),
            pltpu.VMEM((2, 1, CH), jnp.int32),
            pltpu.VMEM((2, 1, CH), jnp.int32),
            pltpu.VMEM((2, 1, CH), jnp.int32),
            pltpu.VMEM((2, CH, HID), jnp.float32),
            pltpu.SemaphoreType.DMA((2,)),
        ],
        compiler_params=pltpu.CompilerParams(use_tc_tiling_on_sc=False),
    )
    return f(u, src, dst)


def _sc_counts(dst):
    mesh = plsc.VectorSubcoreMesh(core_axis_name="c", subcore_axis_name="s")

    def body(dst_hbm, out_hbm, acc, dstb, dstl, ones):
        c = lax.axis_index("c")
        s = lax.axis_index("s")
        base = c * HALF
        trash = HALF + s

        def fill(r, _):
            ones[0, r, pl.ds(0, 16)] = jnp.zeros((16,), jnp.float32)
            ones[1, r, pl.ds(0, 16)] = jnp.ones((16,), jnp.float32)
            return 0

        lax.fori_loop(0, CH, fill, 0)
        row0 = s * RPT
        for j in range(12):
            pltpu.sync_copy(ones.at[0], acc.at[pl.ds(row0 + j * CH, CH)])
        pltpu.sync_copy(ones.at[0, pl.ds(0, 32)], acc.at[pl.ds(row0 + 12 * CH, 32)])
        plsc.subcore_barrier()

        def chunk(i, nreal):
            off = s * EPT + i * CH
            pltpu.sync_copy(dst_hbm.at[pl.ds(off, nreal)], dstb.at[0, 0, pl.ds(0, nreal)])
            for j in range(nreal // 16):
                d = dstb[0, 0, pl.ds(j * 16, 16)]
                dl = d - base
                ok = (dl >= 0) & (dl < HALF)
                dstl[0, 0, pl.ds(j * 16, 16)] = jnp.where(ok, dl, trash)
            for j in range(nreal // 16, 8):
                dstl[0, 0, pl.ds(j * 16, 16)] = jnp.full((16,), trash, jnp.int32)
            pltpu.sync_copy(ones.at[1], acc.at[dstl.at[0, 0]], add=True)

        def loop(i, _):
            chunk(i, CH)
            return 0

        lax.fori_loop(0, NFULL, loop, 0)
        chunk(NFULL, TAIL)

        plsc.subcore_barrier()
        pltpu.sync_copy(acc.at[pl.ds(row0, RPT)], out_hbm.at[c, pl.ds(row0, RPT)])

    f = pl.kernel(
        body,
        out_type=jax.ShapeDtypeStruct((2, HPAD, 16), jnp.float32),
        mesh=mesh,
        scratch_types=[
            pltpu.VMEM_SHARED((HPAD, 16), jnp.float32),
            pltpu.VMEM((2, 1, CH), jnp.int32),
            pltpu.VMEM((2, 1, CH), jnp.int32),
            pltpu.VMEM((2, CH, 16), jnp.float32),
        ],
        compiler_params=pltpu.CompilerParams(use_tc_tiling_on_sc=False),
    )
    return f(dst)


BR = 1000
NBH = HALF // BR


def _tc_pre(h, wlt, wrt, bl):
    k = h.shape[1]

    def body(h_ref, wl_ref, wr_ref, bl_ref, u_ref, v_ref):
        hb = h_ref[...]
        u_ref[...] = jnp.dot(hb, wl_ref[...], preferred_element_type=jnp.float32,
                             precision=lax.Precision.HIGHEST)
        v_ref[...] = (
            jnp.dot(hb, wr_ref[...], preferred_element_type=jnp.float32) + bl_ref[...]
        )

    return pl.pallas_call(
        body,
        grid=(N // BR,),
        in_specs=[
            pl.BlockSpec((BR, k), lambda i: (i, 0)),
            pl.BlockSpec((k, HID), lambda i: (0, 0)),
            pl.BlockSpec((k, HID), lambda i: (0, 0)),
            pl.BlockSpec((1, HID), lambda i: (0, 0)),
        ],
        out_specs=[
            pl.BlockSpec((BR, HID), lambda i: (i, 0)),
            pl.BlockSpec((BR, HID), lambda i: (i, 0)),
        ],
        out_shape=[jax.ShapeDtypeStruct((N, HID), jnp.float32)] * 2,
    )(h, wlt, wrt, bl)


def _tc_post1(s2, cnt2, v, g, b):
    def body(s_ref, c_ref, v_ref, g_ref, b_ref, o_ref):
        cnt = c_ref[0][:, 0:1]
        f = s_ref[0] / jnp.maximum(cnt, 1.0) + v_ref[...]
        o_ref[...] = _layernorm(_gelu(f), g_ref[...], b_ref[...])

    return pl.pallas_call(
        body,
        grid=(2, NBH),
        in_specs=[
            pl.BlockSpec((1, BR, HID), lambda c, i: (c, i, 0)),
            pl.BlockSpec((1, BR, 16), lambda c, i: (c, i, 0)),
            pl.BlockSpec((BR, HID), lambda c, i: (c * NBH + i, 0)),
            pl.BlockSpec((1, HID), lambda c, i: (0, 0)),
            pl.BlockSpec((1, HID), lambda c, i: (0, 0)),
        ],
        out_specs=pl.BlockSpec((BR, HID), lambda c, i: (c * NBH + i, 0)),
        out_shape=jax.ShapeDtypeStruct((N, HID), jnp.float32),
    )(s2, cnt2, v, g, b)


def _tc_layer(s2, cnt2, h, wlt, wrt, bl, g, b):
    def body(s_ref, c_ref, h_ref, wl_ref, wr_ref, bl_ref, g_ref, b_ref, o_ref):
        cnt = c_ref[0][:, 0:1]
        agg = s_ref[0] / jnp.maximum(cnt, 1.0)
        hb = h_ref[...]
        f = (
            jnp.dot(agg, wl_ref[...], preferred_element_type=jnp.float32)
            + bl_ref[...]
            + jnp.dot(hb, wr_ref[...], preferred_element_type=jnp.float32)
        )
        f = _layernorm(_gelu(f), g_ref[...], b_ref[...])
        o_ref[...] = f + hb

    return pl.pallas_call(
        body,
        grid=(2, NBH),
        in_specs=[
            pl.BlockSpec((1, BR, HID), lambda c, i: (c, i, 0)),
            pl.BlockSpec((1, BR, 16), lambda c, i: (c, i, 0)),
            pl.BlockSpec((BR, HID), lambda c, i: (c * NBH + i, 0)),
            pl.BlockSpec((HID, HID), lambda c, i: (0, 0)),
            pl.BlockSpec((HID, HID), lambda c, i: (0, 0)),
            pl.BlockSpec((1, HID), lambda c, i: (0, 0)),
            pl.BlockSpec((1, HID), lambda c, i: (0, 0)),
            pl.BlockSpec((1, HID), lambda c, i: (0, 0)),
        ],
        out_specs=pl.BlockSpec((BR, HID), lambda c, i: (c * NBH + i, 0)),
        out_shape=jax.ShapeDtypeStruct((N, HID), jnp.float32),
    )(s2, cnt2, h, wlt, wrt, bl, g, b)


def _tc_pool(h, batch3):
    PB = 1000

    def body(h_ref, b_ref, o_ref):
        i = pl.program_id(0)

        @pl.when(i == 0)
        def _():
            o_ref[...] = jnp.zeros_like(o_ref)

        ids = b_ref[0, 0, :]
        oh = (lax.broadcasted_iota(jnp.int32, (G, PB), 0) == ids[None, :]).astype(
            jnp.float32
        )
        o_ref[...] += jnp.dot(oh, h_ref[...], preferred_element_type=jnp.float32,
                              precision=lax.Precision.HIGHEST)

    return pl.pallas_call(
        body,
        grid=(N // PB,),
        in_specs=[
            pl.BlockSpec((PB, HID), lambda i: (i, 0)),
            pl.BlockSpec((1, 1, PB), lambda i: (i, 0, 0)),
        ],
        out_specs=pl.BlockSpec((G, HID), lambda i: (0, 0)),
        out_shape=jax.ShapeDtypeStruct((G, HID), jnp.float32),
    )(h, batch3)


def _tc_mlp(pool, gf, w0a, w0b, b0, g0, lb0, w1, b1, g1, lb1, w2, b2):
    def body(p_ref, gf_ref, w0a_ref, w0b_ref, b0_ref, g0_ref, lb0_ref, w1_ref,
             b1_ref, g1_ref, lb1_ref, w2_ref, b2_ref, o_ref):
        f = (
            jnp.dot(p_ref[...], w0a_ref[...], preferred_element_type=jnp.float32)
            + jnp.dot(gf_ref[...], w0b_ref[...], preferred_element_type=jnp.float32)
            + b0_ref[...]
        )
        f = _layernorm(_gelu(f), g0_ref[...], lb0_ref[...])
        f2 = jnp.dot(f, w1_ref[...], preferred_element_type=jnp.float32) + b1_ref[...]
        f2 = _layernorm(_gelu(f2), g1_ref[...], lb1_ref[...]) + f
        o_ref[...] = jnp.sum(f2 * w2_ref[...], axis=1, keepdims=True) + b2_ref[...]

    return pl.pallas_call(
        body,
        out_shape=jax.ShapeDtypeStruct((G, 1), jnp.float32),
    )(pool, gf, w0a, w0b, b0, g0, lb0, w1, b1, g1, lb1, w2, b2)


def kernel(x, edge_index, batch, global_features, params):
    src = edge_index[0]
    dst = edge_index[1]
    batch3 = batch.reshape(N // 1000, 1, 1000)

    cnt2 = _sc_counts(dst)

    p = params["sage"][0]
    u, v = _tc_pre(x, p["Wl"].T, p["Wr"].T, p["bl"].reshape(1, HID))
    s2 = _sc_segsum(u, src, dst)
    h = _tc_post1(s2, cnt2, v, p["g"].reshape(1, HID), p["b"].reshape(1, HID))

    for p in params["sage"][1:]:
        s2 = _sc_segsum(h, src, dst)
        h = _tc_layer(
            s2, cnt2, h, p["Wl"].T, p["Wr"].T, p["bl"].reshape(1, HID),
            p["g"].reshape(1, HID), p["b"].reshape(1, HID),
        )

    pool = _tc_pool(h, batch3)

    m = params["mlp"]
    w0 = m["W0"].T
    return _tc_mlp(
        pool,
        global_features,
        w0[:HID],
        w0[HID:],
        m["b0"].reshape(1, 32),
        m["g0"].reshape(1, 32),
        m["lb0"].reshape(1, 32),
        m["W1"].T,
        m["b1"].reshape(1, 32),
        m["g1"].reshape(1, 32),
        m["lb1"].reshape(1, 32),
        m["W2"].reshape(1, 32),
        m["b2"].reshape(1, 1),
    )


# speedup vs baseline: 5.4784x; 1.0001x over previous
import math

import jax
import jax.numpy as jnp
from jax import lax
from jax.experimental import pallas as pl
from jax.experimental.pallas import tpu as pltpu
from jax.experimental.pallas import tpu_sc as plsc

N = 50000
E = 800000
G = 128
HID = 64
HALF = 25000
HPAD = 25088
RPT = HPAD // 16
EPT = E // 16
CH = 128
NFULL = EPT // CH
TAIL = EPT - NFULL * CH

_INV_SQRT2 = 1.0 / math.sqrt(2.0)


def _gelu(f):
    return 0.5 * f * (1.0 + lax.erf(f * _INV_SQRT2))


def _layernorm(f, g, b):
    mu = jnp.mean(f, axis=-1, keepdims=True)
    var = jnp.mean((f - mu) ** 2, axis=-1, keepdims=True)
    return (f - mu) / jnp.sqrt(var + 1e-5) * g + b


def _sc_segsum(u, src, dst):
    mesh = plsc.VectorSubcoreMesh(core_axis_name="c", subcore_axis_name="s")

    def body(u_hbm, src_hbm, dst_hbm, out_hbm, acc, srcb, dstb, dstl, rows, sems):
        c = lax.axis_index("c")
        s = lax.axis_index("s")
        base = c * HALF
        trash = HALF + s

        def zr(r, _):
            for k in range(4):
                rows[0, r, pl.ds(k * 16, 16)] = jnp.zeros((16,), jnp.float32)
            return 0

        lax.fori_loop(0, CH, zr, 0)
        row0 = s * RPT
        for j in range(12):
            pltpu.sync_copy(rows.at[0], acc.at[pl.ds(row0 + j * CH, CH)])
        pltpu.sync_copy(rows.at[0, pl.ds(0, 32)], acc.at[pl.ds(row0 + 12 * CH, 32)])
        plsc.subcore_barrier()

        def load_idx(i, b):
            off = s * EPT + i * CH
            pltpu.sync_copy(src_hbm.at[pl.ds(off, CH)], srcb.at[b, 0])
            pltpu.sync_copy(dst_hbm.at[pl.ds(off, CH)], dstb.at[b, 0])

        def start_gather(b):
            pltpu.make_async_copy(u_hbm.at[srcb.at[b, 0]], rows.at[b], sems.at[b]).start()

        def wait_gather(b):
            pltpu.make_async_copy(u_hbm.at[srcb.at[b, 0]], rows.at[b], sems.at[b]).wait()

        def compute_dstl(b):
            for j in range(8):
                d = dstb[b, 0, pl.ds(j * 16, 16)]
                dl = d - base
                ok = (dl >= 0) & (dl < HALF)
                dstl[b, 0, pl.ds(j * 16, 16)] = jnp.where(ok, dl, trash)

        def scatter(b):
            pltpu.sync_copy(rows.at[b], acc.at[dstl.at[b, 0]], add=True)

        load_idx(0, 0)
        start_gather(0)

        def ring(k, _):
            for b in range(2):
                i = 2 * k + b

                @pl.when(i + 1 < NFULL)
                def _():
                    load_idx(i + 1, 1 - b)
                    start_gather(1 - b)

                compute_dstl(b)
                wait_gather(b)
                scatter(b)
            return 0

        lax.fori_loop(0, NFULL // 2, ring, 0)

        off = s * EPT + NFULL * CH
        pltpu.sync_copy(src_hbm.at[pl.ds(off, TAIL)], srcb.at[0, 0, pl.ds(0, TAIL)])
        pltpu.sync_copy(dst_hbm.at[pl.ds(off, TAIL)], dstb.at[0, 0, pl.ds(0, TAIL)])
        for j in range(TAIL // 16):
            d = dstb[0, 0, pl.ds(j * 16, 16)]
            dl = d - base
            ok = (dl >= 0) & (dl < HALF)
            dstl[0, 0, pl.ds(j * 16, 16)] = jnp.where(ok, dl, trash)
        for j in range(TAIL // 16, 8):
            dstl[0, 0, pl.ds(j * 16, 16)] = jnp.full((16,), trash, jnp.int32)
        pltpu.make_async_copy(u_hbm.at[srcb.at[0, 0]], rows.at[0], sems.at[0]).start()
        wait_gather(0)
        scatter(0)

        plsc.subcore_barrier()
        pltpu.sync_copy(acc.at[pl.ds(row0, RPT)], out_hbm.at[c, pl.ds(row0, RPT)])

    f = pl.kernel(
        body,
        out_type=jax.ShapeDtypeStruct((2, HPAD, HID), jnp.float32),
        mesh=mesh,
        scratch_types=[
            pltpu.VMEM_SHARED((HPAD, HID), jnp.float32),
            pltpu.VMEM((2, 1, CH), jnp.int32),
            pltpu.VMEM((2, 1, CH), jnp.int32),
            pltpu.VMEM((2, 1, CH), jnp.int32),
            pltpu.VMEM((2, CH, HID), jnp.float32),
            pltpu.SemaphoreType.DMA((2,)),
        ],
        compiler_params=pltpu.CompilerParams(use_tc_tiling_on_sc=False),
    )
    return f(u, src, dst)


CAP = 50176
NCH_MAX = CAP // CH


def _sc_partition(src, dst):
    mesh = plsc.VectorSubcoreMesh(core_axis_name="c", subcore_axis_name="s")

    def body(src_hbm, dst_hbm, esrc_hbm, edstl_hbm, ecnt_hbm,
             sbuf, dbuf, sb, db, cb):
        c = lax.axis_index("c")
        s = lax.axis_index("s")
        wid = c * 16 + s
        base = c * HALF
        trash = HALF + s

        def chunk(i, nreal, pos):
            off = s * EPT + i * CH
            pltpu.sync_copy(src_hbm.at[pl.ds(off, nreal)], sb.at[0, pl.ds(0, nreal)])
            pltpu.sync_copy(dst_hbm.at[pl.ds(off, nreal)], db.at[0, pl.ds(0, nreal)])
            for j in range(nreal // 16):
                d = db[0, pl.ds(j * 16, 16)]
                sv = sb[0, pl.ds(j * 16, 16)]
                dl = d - base
                ok = (dl >= 0) & (dl < HALF)
                oki = ok.astype(jnp.int32)
                cs = plsc.cumsum(oki)
                idx = pos + cs - oki
                plsc.store_scatter(sbuf, [idx], sv, mask=ok)
                plsc.store_scatter(dbuf, [idx], jnp.where(ok, dl, trash), mask=ok)
                pos = pos + plsc.all_reduce_population_count(ok)
            return pos

        pos0 = jnp.zeros((16,), jnp.int32)
        pos = lax.fori_loop(0, NFULL, lambda i, p: chunk(i, CH, p), pos0)
        pos = chunk(NFULL, TAIL, pos)

        lane = lax.iota(jnp.int32, 16)
        for k in range(8):
            idx = pos + k * 16 + lane
            plsc.store_scatter(sbuf, [idx], jnp.zeros((16,), jnp.int32))
            plsc.store_scatter(dbuf, [idx], jnp.full((16,), trash, jnp.int32))
        cb[pl.ds(0, 16)] = (pos + (CH - 1)) // CH

        pltpu.sync_copy(sbuf, esrc_hbm.at[wid])
        pltpu.sync_copy(dbuf, edstl_hbm.at[wid])
        pltpu.sync_copy(cb, ecnt_hbm.at[wid])

    f = pl.kernel(
        body,
        out_type=(
            jax.ShapeDtypeStruct((32, CAP), jnp.int32),
            jax.ShapeDtypeStruct((32, CAP), jnp.int32),
            jax.ShapeDtypeStruct((32, 16), jnp.int32),
        ),
        mesh=mesh,
        scratch_types=[
            pltpu.VMEM((CAP,), jnp.int32),
            pltpu.VMEM((CAP,), jnp.int32),
            pltpu.VMEM((1, CH), jnp.int32),
            pltpu.VMEM((1, CH), jnp.int32),
            pltpu.VMEM((16,), jnp.int32),
        ],
        compiler_params=pltpu.CompilerParams(use_tc_tiling_on_sc=False),
    )
    return f(src, dst)


def _sc_segsum2(u, esrc, edstl, ecnt):
    mesh = plsc.VectorSubcoreMesh(core_axis_name="c", subcore_axis_name="s")

    def body(u_hbm, esrc_hbm, edstl_hbm, ecnt_hbm, out_hbm,
             acc, srcb, dstlb, rows, cb, sems):
        c = lax.axis_index("c")
        s = lax.axis_index("s")
        wid = c * 16 + s
        row0 = s * RPT

        def zr(r, _):
            for k in range(4):
                rows[0, r, pl.ds(k * 16, 16)] = jnp.zeros((16,), jnp.float32)
            return 0

        lax.fori_loop(0, CH, zr, 0)
        for j in range(12):
            pltpu.sync_copy(rows.at[0], acc.at[pl.ds(row0 + j * CH, CH)])
        pltpu.sync_copy(rows.at[0, pl.ds(0, 32)], acc.at[pl.ds(row0 + 12 * CH, 32)])

        pltpu.sync_copy(ecnt_hbm.at[wid], cb)
        nch = jnp.max(cb[...])
        plsc.subcore_barrier()

        def load_start(i, b):
            pltpu.sync_copy(esrc_hbm.at[wid, pl.ds(i * CH, CH)], srcb.at[b, 0])
            pltpu.sync_copy(edstl_hbm.at[wid, pl.ds(i * CH, CH)], dstlb.at[b, 0])
            pltpu.make_async_copy(
                u_hbm.at[srcb.at[b, 0]], rows.at[b], sems.at[b]).start()

        @pl.when(nch > 0)
        def _():
            load_start(0, 0)

        def ring(k, _):
            for b in range(2):
                i = 2 * k + b

                @pl.when(i + 1 < nch)
                def _():
                    load_start(i + 1, 1 - b)

                @pl.when(i < nch)
                def _():
                    pltpu.make_async_copy(
                        u_hbm.at[srcb.at[b, 0]], rows.at[b], sems.at[b]).wait()
                    pltpu.sync_copy(rows.at[b], acc.at[dstlb.at[b, 0]], add=True)
            return 0

        lax.fori_loop(0, (nch + 1) // 2, ring, 0)

        plsc.subcore_barrier()
        pltpu.sync_copy(acc.at[pl.ds(row0, RPT)], out_hbm.at[c, pl.ds(row0, RPT)])

    f = pl.kernel(
        body,
        out_type=jax.ShapeDtypeStruct((2, HPAD, HID), jnp.float32),
        mesh=mesh,
        scratch_types=[
            pltpu.VMEM_SHARED((HPAD, HID), jnp.float32),
            pltpu.VMEM((2, 1, CH), jnp.int32),
            pltpu.VMEM((2, 1, CH), jnp.int32),
            pltpu.VMEM((2, CH, HID), jnp.float32),
            pltpu.VMEM((16,), jnp.int32),
            pltpu.SemaphoreType.DMA((2,)),
        ],
        compiler_params=pltpu.CompilerParams(use_tc_tiling_on_sc=False),
    )
    return f(u, esrc, edstl, ecnt)


def _sc_counts(dst):
    mesh = plsc.VectorSubcoreMesh(core_axis_name="c", subcore_axis_name="s")

    def body(dst_hbm, out_hbm, acc, dstb, dstl, ones):
        c = lax.axis_index("c")
        s = lax.axis_index("s")
        base = c * HALF
        trash = HALF + s

        def fill(r, _):
            ones[0, r, pl.ds(0, 16)] = jnp.zeros((16,), jnp.float32)
            ones[1, r, pl.ds(0, 16)] = jnp.ones((16,), jnp.float32)
            return 0

        lax.fori_loop(0, CH, fill, 0)
        row0 = s * RPT
        for j in range(12):
            pltpu.sync_copy(ones.at[0], acc.at[pl.ds(row0 + j * CH, CH)])
        pltpu.sync_copy(ones.at[0, pl.ds(0, 32)], acc.at[pl.ds(row0 + 12 * CH, 32)])
        plsc.subcore_barrier()

        def chunk(i, nreal):
            off = s * EPT + i * CH
            pltpu.sync_copy(dst_hbm.at[pl.ds(off, nreal)], dstb.at[0, 0, pl.ds(0, nreal)])
            for j in range(nreal // 16):
                d = dstb[0, 0, pl.ds(j * 16, 16)]
                dl = d - base
                ok = (dl >= 0) & (dl < HALF)
                dstl[0, 0, pl.ds(j * 16, 16)] = jnp.where(ok, dl, trash)
            for j in range(nreal // 16, 8):
                dstl[0, 0, pl.ds(j * 16, 16)] = jnp.full((16,), trash, jnp.int32)
            pltpu.sync_copy(ones.at[1], acc.at[dstl.at[0, 0]], add=True)

        def loop(i, _):
            chunk(i, CH)
            return 0

        lax.fori_loop(0, NFULL, loop, 0)
        chunk(NFULL, TAIL)

        plsc.subcore_barrier()
        pltpu.sync_copy(acc.at[pl.ds(row0, RPT)], out_hbm.at[c, pl.ds(row0, RPT)])

    f = pl.kernel(
        body,
        out_type=jax.ShapeDtypeStruct((2, HPAD, 16), jnp.float32),
        mesh=mesh,
        scratch_types=[
            pltpu.VMEM_SHARED((HPAD, 16), jnp.float32),
            pltpu.VMEM((2, 1, CH), jnp.int32),
            pltpu.VMEM((2, 1, CH), jnp.int32),
            pltpu.VMEM((2, CH, 16), jnp.float32),
        ],
        compiler_params=pltpu.CompilerParams(use_tc_tiling_on_sc=False),
    )
    return f(dst)


BR = 1000
NBH = HALF // BR


def _tc_pre(h, wlt, wrt, bl):
    k = h.shape[1]

    def body(h_ref, wl_ref, wr_ref, bl_ref, u_ref, v_ref):
        hb = h_ref[...]
        u_ref[...] = jnp.dot(hb, wl_ref[...], preferred_element_type=jnp.float32,
                             precision=lax.Precision.HIGHEST)
        v_ref[...] = (
            jnp.dot(hb, wr_ref[...], preferred_element_type=jnp.float32) + bl_ref[...]
        )

    return pl.pallas_call(
        body,
        grid=(N // BR,),
        in_specs=[
            pl.BlockSpec((BR, k), lambda i: (i, 0)),
            pl.BlockSpec((k, HID), lambda i: (0, 0)),
            pl.BlockSpec((k, HID), lambda i: (0, 0)),
            pl.BlockSpec((1, HID), lambda i: (0, 0)),
        ],
        out_specs=[
            pl.BlockSpec((BR, HID), lambda i: (i, 0)),
            pl.BlockSpec((BR, HID), lambda i: (i, 0)),
        ],
        out_shape=[jax.ShapeDtypeStruct((N, HID), jnp.float32)] * 2,
    )(h, wlt, wrt, bl)


def _tc_post1(s2, cnt2, v, g, b):
    def body(s_ref, c_ref, v_ref, g_ref, b_ref, o_ref):
        cnt = c_ref[0][:, 0:1]
        f = s_ref[0] / jnp.maximum(cnt, 1.0) + v_ref[...]
        o_ref[...] = _layernorm(_gelu(f), g_ref[...], b_ref[...])

    return pl.pallas_call(
        body,
        grid=(2, NBH),
        in_specs=[
            pl.BlockSpec((1, BR, HID), lambda c, i: (c, i, 0)),
            pl.BlockSpec((1, BR, 16), lambda c, i: (c, i, 0)),
            pl.BlockSpec((BR, HID), lambda c, i: (c * NBH + i, 0)),
            pl.BlockSpec((1, HID), lambda c, i: (0, 0)),
            pl.BlockSpec((1, HID), lambda c, i: (0, 0)),
        ],
        out_specs=pl.BlockSpec((BR, HID), lambda c, i: (c * NBH + i, 0)),
        out_shape=jax.ShapeDtypeStruct((N, HID), jnp.float32),
    )(s2, cnt2, v, g, b)


def _tc_layer(s2, cnt2, h, wlt, wrt, bl, g, b):
    def body(s_ref, c_ref, h_ref, wl_ref, wr_ref, bl_ref, g_ref, b_ref, o_ref):
        cnt = c_ref[0][:, 0:1]
        agg = s_ref[0] / jnp.maximum(cnt, 1.0)
        hb = h_ref[...]
        f = (
            jnp.dot(agg, wl_ref[...], preferred_element_type=jnp.float32)
            + bl_ref[...]
            + jnp.dot(hb, wr_ref[...], preferred_element_type=jnp.float32)
        )
        f = _layernorm(_gelu(f), g_ref[...], b_ref[...])
        o_ref[...] = f + hb

    return pl.pallas_call(
        body,
        grid=(2, NBH),
        in_specs=[
            pl.BlockSpec((1, BR, HID), lambda c, i: (c, i, 0)),
            pl.BlockSpec((1, BR, 16), lambda c, i: (c, i, 0)),
            pl.BlockSpec((BR, HID), lambda c, i: (c * NBH + i, 0)),
            pl.BlockSpec((HID, HID), lambda c, i: (0, 0)),
            pl.BlockSpec((HID, HID), lambda c, i: (0, 0)),
            pl.BlockSpec((1, HID), lambda c, i: (0, 0)),
            pl.BlockSpec((1, HID), lambda c, i: (0, 0)),
            pl.BlockSpec((1, HID), lambda c, i: (0, 0)),
        ],
        out_specs=pl.BlockSpec((BR, HID), lambda c, i: (c * NBH + i, 0)),
        out_shape=jax.ShapeDtypeStruct((N, HID), jnp.float32),
    )(s2, cnt2, h, wlt, wrt, bl, g, b)


def _tc_pool(h, batch3):
    PB = 1000

    def body(h_ref, b_ref, o_ref):
        i = pl.program_id(0)

        @pl.when(i == 0)
        def _():
            o_ref[...] = jnp.zeros_like(o_ref)

        ids = b_ref[0, 0, :]
        oh = (lax.broadcasted_iota(jnp.int32, (G, PB), 0) == ids[None, :]).astype(
            jnp.float32
        )
        o_ref[...] += jnp.dot(oh, h_ref[...], preferred_element_type=jnp.float32,
                              precision=lax.Precision.HIGHEST)

    return pl.pallas_call(
        body,
        grid=(N // PB,),
        in_specs=[
            pl.BlockSpec((PB, HID), lambda i: (i, 0)),
            pl.BlockSpec((1, 1, PB), lambda i: (i, 0, 0)),
        ],
        out_specs=pl.BlockSpec((G, HID), lambda i: (0, 0)),
        out_shape=jax.ShapeDtypeStruct((G, HID), jnp.float32),
    )(h, batch3)


def _tc_mlp(pool, gf, w0a, w0b, b0, g0, lb0, w1, b1, g1, lb1, w2, b2):
    def body(p_ref, gf_ref, w0a_ref, w0b_ref, b0_ref, g0_ref, lb0_ref, w1_ref,
             b1_ref, g1_ref, lb1_ref, w2_ref, b2_ref, o_ref):
        f = (
            jnp.dot(p_ref[...], w0a_ref[...], preferred_element_type=jnp.float32)
            + jnp.dot(gf_ref[...], w0b_ref[...], preferred_element_type=jnp.float32)
            + b0_ref[...]
        )
        f = _layernorm(_gelu(f), g0_ref[...], lb0_ref[...])
        f2 = jnp.dot(f, w1_ref[...], preferred_element_type=jnp.float32) + b1_ref[...]
        f2 = _layernorm(_gelu(f2), g1_ref[...], lb1_ref[...]) + f
        o_ref[...] = jnp.sum(f2 * w2_ref[...], axis=1, keepdims=True) + b2_ref[...]

    return pl.pallas_call(
        body,
        out_shape=jax.ShapeDtypeStruct((G, 1), jnp.float32),
    )(pool, gf, w0a, w0b, b0, g0, lb0, w1, b1, g1, lb1, w2, b2)


def kernel(x, edge_index, batch, global_features, params):
    src = edge_index[0]
    dst = edge_index[1]
    batch3 = batch.reshape(N // 1000, 1, 1000)

    cnt2 = _sc_counts(dst)

    p = params["sage"][0]
    u, v = _tc_pre(x, p["Wl"].T, p["Wr"].T, p["bl"].reshape(1, HID))
    s2 = _sc_segsum(u, src, dst)
    h = _tc_post1(s2, cnt2, v, p["g"].reshape(1, HID), p["b"].reshape(1, HID))

    for p in params["sage"][1:]:
        s2 = _sc_segsum(h, src, dst)
        h = _tc_layer(
            s2, cnt2, h, p["Wl"].T, p["Wr"].T, p["bl"].reshape(1, HID),
            p["g"].reshape(1, HID), p["b"].reshape(1, HID),
        )

    pool = _tc_pool(h, batch3)

    m = params["mlp"]
    w0 = m["W0"].T
    return _tc_mlp(
        pool,
        global_features,
        w0[:HID],
        w0[HID:],
        m["b0"].reshape(1, 32),
        m["g0"].reshape(1, 32),
        m["lb0"].reshape(1, 32),
        m["W1"].T,
        m["b1"].reshape(1, 32),
        m["g1"].reshape(1, 32),
        m["lb1"].reshape(1, 32),
        m["W2"].reshape(1, 32),
        m["b2"].reshape(1, 1),
    )


# speedup vs baseline: 7.4156x; 1.3536x over previous
import math

import jax
import jax.numpy as jnp
from jax import lax
from jax.experimental import pallas as pl
from jax.experimental.pallas import tpu as pltpu
from jax.experimental.pallas import tpu_sc as plsc

N = 50000
E = 800000
G = 128
HID = 64
HALF = 25000
HPAD = 25088
RPT = HPAD // 16
EPT = E // 16
CH = 128
NFULL = EPT // CH
TAIL = EPT - NFULL * CH

_INV_SQRT2 = 1.0 / math.sqrt(2.0)


def _gelu(f):
    return 0.5 * f * (1.0 + lax.erf(f * _INV_SQRT2))


def _layernorm(f, g, b):
    mu = jnp.mean(f, axis=-1, keepdims=True)
    var = jnp.mean((f - mu) ** 2, axis=-1, keepdims=True)
    return (f - mu) / jnp.sqrt(var + 1e-5) * g + b


def _sc_segsum(u, src, dst):
    mesh = plsc.VectorSubcoreMesh(core_axis_name="c", subcore_axis_name="s")

    def body(u_hbm, src_hbm, dst_hbm, out_hbm, acc, srcb, dstb, dstl, rows, sems):
        c = lax.axis_index("c")
        s = lax.axis_index("s")
        base = c * HALF
        trash = HALF + s

        def zr(r, _):
            for k in range(4):
                rows[0, r, pl.ds(k * 16, 16)] = jnp.zeros((16,), jnp.float32)
            return 0

        lax.fori_loop(0, CH, zr, 0)
        row0 = s * RPT
        for j in range(12):
            pltpu.sync_copy(rows.at[0], acc.at[pl.ds(row0 + j * CH, CH)])
        pltpu.sync_copy(rows.at[0, pl.ds(0, 32)], acc.at[pl.ds(row0 + 12 * CH, 32)])
        plsc.subcore_barrier()

        def load_idx(i, b):
            off = s * EPT + i * CH
            pltpu.sync_copy(src_hbm.at[pl.ds(off, CH)], srcb.at[b, 0])
            pltpu.sync_copy(dst_hbm.at[pl.ds(off, CH)], dstb.at[b, 0])

        def start_gather(b):
            pltpu.make_async_copy(u_hbm.at[srcb.at[b, 0]], rows.at[b], sems.at[b]).start()

        def wait_gather(b):
            pltpu.make_async_copy(u_hbm.at[srcb.at[b, 0]], rows.at[b], sems.at[b]).wait()

        def compute_dstl(b):
            for j in range(8):
                d = dstb[b, 0, pl.ds(j * 16, 16)]
                dl = d - base
                ok = (dl >= 0) & (dl < HALF)
                dstl[b, 0, pl.ds(j * 16, 16)] = jnp.where(ok, dl, trash)

        def scatter(b):
            pltpu.sync_copy(rows.at[b], acc.at[dstl.at[b, 0]], add=True)

        load_idx(0, 0)
        start_gather(0)

        def ring(k, _):
            for b in range(2):
                i = 2 * k + b

                @pl.when(i + 1 < NFULL)
                def _():
                    load_idx(i + 1, 1 - b)
                    start_gather(1 - b)

                compute_dstl(b)
                wait_gather(b)
                scatter(b)
            return 0

        lax.fori_loop(0, NFULL // 2, ring, 0)

        off = s * EPT + NFULL * CH
        pltpu.sync_copy(src_hbm.at[pl.ds(off, TAIL)], srcb.at[0, 0, pl.ds(0, TAIL)])
        pltpu.sync_copy(dst_hbm.at[pl.ds(off, TAIL)], dstb.at[0, 0, pl.ds(0, TAIL)])
        for j in range(TAIL // 16):
            d = dstb[0, 0, pl.ds(j * 16, 16)]
            dl = d - base
            ok = (dl >= 0) & (dl < HALF)
            dstl[0, 0, pl.ds(j * 16, 16)] = jnp.where(ok, dl, trash)
        for j in range(TAIL // 16, 8):
            dstl[0, 0, pl.ds(j * 16, 16)] = jnp.full((16,), trash, jnp.int32)
        pltpu.make_async_copy(u_hbm.at[srcb.at[0, 0]], rows.at[0], sems.at[0]).start()
        wait_gather(0)
        scatter(0)

        plsc.subcore_barrier()
        pltpu.sync_copy(acc.at[pl.ds(row0, RPT)], out_hbm.at[c, pl.ds(row0, RPT)])

    f = pl.kernel(
        body,
        out_type=jax.ShapeDtypeStruct((2, HPAD, HID), jnp.float32),
        mesh=mesh,
        scratch_types=[
            pltpu.VMEM_SHARED((HPAD, HID), jnp.float32),
            pltpu.VMEM((2, 1, CH), jnp.int32),
            pltpu.VMEM((2, 1, CH), jnp.int32),
            pltpu.VMEM((2, 1, CH), jnp.int32),
            pltpu.VMEM((2, CH, HID), jnp.float32),
            pltpu.SemaphoreType.DMA((2,)),
        ],
        compiler_params=pltpu.CompilerParams(use_tc_tiling_on_sc=False),
    )
    return f(u, src, dst)


CAP = 50176
NCH_MAX = CAP // CH


def _sc_partition(src, dst):
    mesh = plsc.VectorSubcoreMesh(core_axis_name="c", subcore_axis_name="s")

    def body(src_hbm, dst_hbm, esrc_hbm, edstl_hbm, ecnt_hbm,
             sbuf, dbuf, sb, db, cb):
        c = lax.axis_index("c")
        s = lax.axis_index("s")
        wid = c * 16 + s
        base = c * HALF
        trash = HALF + s

        lane = lax.iota(jnp.int32, 16)

        def chunk(i, nreal, pos):
            off = s * EPT + i * CH
            pltpu.sync_copy(src_hbm.at[pl.ds(off, nreal)], sb.at[0, pl.ds(0, nreal)])
            pltpu.sync_copy(dst_hbm.at[pl.ds(off, nreal)], db.at[0, pl.ds(0, nreal)])
            for j in range(nreal // 16):
                d = db[0, pl.ds(j * 16, 16)]
                sv = sb[0, pl.ds(j * 16, 16)]
                dl = d - base
                ok = (dl >= 0) & (dl < HALF)
                oki = ok.astype(jnp.int32)
                cs = plsc.cumsum(oki)
                idx = pos + cs - oki
                plsc.store_scatter(sbuf, [idx], sv, mask=ok)
                plsc.store_scatter(dbuf, [idx], jnp.where(ok, dl, trash), mask=ok)
                pos = pos + plsc.all_reduce_population_count(ok)
            return pos

        pos = lax.fori_loop(0, NFULL, lambda i, p: chunk(i, CH, p),
                            jnp.zeros((16,), jnp.int32))
        pos = chunk(NFULL, TAIL, pos)

        for k in range(8):
            idx = pos + k * 16 + lane
            plsc.store_scatter(sbuf, [idx], jnp.zeros((16,), jnp.int32))
            plsc.store_scatter(dbuf, [idx], jnp.full((16,), trash, jnp.int32))
        cb[pl.ds(0, 16)] = (pos + (CH - 1)) // CH

        pltpu.sync_copy(sbuf, esrc_hbm.at[wid])
        pltpu.sync_copy(dbuf, edstl_hbm.at[wid])
        pltpu.sync_copy(cb, ecnt_hbm.at[wid])

    f = pl.kernel(
        body,
        out_type=(
            jax.ShapeDtypeStruct((32, CAP), jnp.int32),
            jax.ShapeDtypeStruct((32, CAP), jnp.int32),
            jax.ShapeDtypeStruct((32, 16), jnp.int32),
        ),
        mesh=mesh,
        scratch_types=[
            pltpu.VMEM((CAP,), jnp.int32),
            pltpu.VMEM((CAP,), jnp.int32),
            pltpu.VMEM((1, CH), jnp.int32),
            pltpu.VMEM((1, CH), jnp.int32),
            pltpu.VMEM((16,), jnp.int32),
        ],
        compiler_params=pltpu.CompilerParams(
            use_tc_tiling_on_sc=False, needs_layout_passes=False),
    )
    return f(src, dst)


def _sc_segsum2(u, esrc, edstl, ecnt):
    mesh = plsc.VectorSubcoreMesh(core_axis_name="c", subcore_axis_name="s")

    def body(u_hbm, esrc_hbm, edstl_hbm, ecnt_hbm, out_hbm,
             acc, srcb, dstlb, rows, cb, sems):
        c = lax.axis_index("c")
        s = lax.axis_index("s")
        wid = c * 16 + s
        row0 = s * RPT

        def zr(r, _):
            for k in range(4):
                rows[0, r, pl.ds(k * 16, 16)] = jnp.zeros((16,), jnp.float32)
            return 0

        lax.fori_loop(0, CH, zr, 0)
        for j in range(12):
            pltpu.sync_copy(rows.at[0], acc.at[pl.ds(row0 + j * CH, CH)])
        pltpu.sync_copy(rows.at[0, pl.ds(0, 32)], acc.at[pl.ds(row0 + 12 * CH, 32)])

        pltpu.sync_copy(ecnt_hbm.at[wid], cb)
        nch = jnp.max(cb[...])
        plsc.subcore_barrier()

        def load_start(i, b):
            pltpu.sync_copy(esrc_hbm.at[wid, pl.ds(i * CH, CH)], srcb.at[b, 0])
            pltpu.sync_copy(edstl_hbm.at[wid, pl.ds(i * CH, CH)], dstlb.at[b, 0])
            pltpu.make_async_copy(
                u_hbm.at[srcb.at[b, 0]], rows.at[b], sems.at[b]).start()

        @pl.when(nch > 0)
        def _():
            load_start(0, 0)

        def ring(k, _):
            for b in range(2):
                i = 2 * k + b

                @pl.when(i + 1 < nch)
                def _():
                    load_start(i + 1, 1 - b)

                @pl.when(i < nch)
                def _():
                    pltpu.make_async_copy(
                        u_hbm.at[srcb.at[b, 0]], rows.at[b], sems.at[b]).wait()
                    pltpu.sync_copy(rows.at[b], acc.at[dstlb.at[b, 0]], add=True)
            return 0

        lax.fori_loop(0, (nch + 1) // 2, ring, 0)

        plsc.subcore_barrier()
        pltpu.sync_copy(acc.at[pl.ds(row0, RPT)], out_hbm.at[c, pl.ds(row0, RPT)])

    f = pl.kernel(
        body,
        out_type=jax.ShapeDtypeStruct((2, HPAD, HID), jnp.float32),
        mesh=mesh,
        scratch_types=[
            pltpu.VMEM_SHARED((HPAD, HID), jnp.float32),
            pltpu.VMEM((2, 1, CH), jnp.int32),
            pltpu.VMEM((2, 1, CH), jnp.int32),
            pltpu.VMEM((2, CH, HID), jnp.float32),
            pltpu.VMEM((16,), jnp.int32),
            pltpu.SemaphoreType.DMA((2,)),
        ],
        compiler_params=pltpu.CompilerParams(
            use_tc_tiling_on_sc=False, needs_layout_passes=False),
    )
    return f(u, esrc, edstl, ecnt)


def _sc_counts(dst):
    mesh = plsc.VectorSubcoreMesh(core_axis_name="c", subcore_axis_name="s")

    def body(dst_hbm, out_hbm, acc, dstb, dstl, ones):
        c = lax.axis_index("c")
        s = lax.axis_index("s")
        base = c * HALF
        trash = HALF + s

        def fill(r, _):
            ones[0, r, pl.ds(0, 16)] = jnp.zeros((16,), jnp.float32)
            ones[1, r, pl.ds(0, 16)] = jnp.ones((16,), jnp.float32)
            return 0

        lax.fori_loop(0, CH, fill, 0)
        row0 = s * RPT
        for j in range(12):
            pltpu.sync_copy(ones.at[0], acc.at[pl.ds(row0 + j * CH, CH)])
        pltpu.sync_copy(ones.at[0, pl.ds(0, 32)], acc.at[pl.ds(row0 + 12 * CH, 32)])
        plsc.subcore_barrier()

        def chunk(i, nreal):
            off = s * EPT + i * CH
            pltpu.sync_copy(dst_hbm.at[pl.ds(off, nreal)], dstb.at[0, 0, pl.ds(0, nreal)])
            for j in range(nreal // 16):
                d = dstb[0, 0, pl.ds(j * 16, 16)]
                dl = d - base
                ok = (dl >= 0) & (dl < HALF)
                dstl[0, 0, pl.ds(j * 16, 16)] = jnp.where(ok, dl, trash)
            for j in range(nreal // 16, 8):
                dstl[0, 0, pl.ds(j * 16, 16)] = jnp.full((16,), trash, jnp.int32)
            pltpu.sync_copy(ones.at[1], acc.at[dstl.at[0, 0]], add=True)

        def loop(i, _):
            chunk(i, CH)
            return 0

        lax.fori_loop(0, NFULL, loop, 0)
        chunk(NFULL, TAIL)

        plsc.subcore_barrier()
        pltpu.sync_copy(acc.at[pl.ds(row0, RPT)], out_hbm.at[c, pl.ds(row0, RPT)])

    f = pl.kernel(
        body,
        out_type=jax.ShapeDtypeStruct((2, HPAD, 16), jnp.float32),
        mesh=mesh,
        scratch_types=[
            pltpu.VMEM_SHARED((HPAD, 16), jnp.float32),
            pltpu.VMEM((2, 1, CH), jnp.int32),
            pltpu.VMEM((2, 1, CH), jnp.int32),
            pltpu.VMEM((2, CH, 16), jnp.float32),
        ],
        compiler_params=pltpu.CompilerParams(use_tc_tiling_on_sc=False),
    )
    return f(dst)


BR = 1000
NBH = HALF // BR


def _tc_pre(h, wlt, wrt, bl):
    k = h.shape[1]

    def body(h_ref, wl_ref, wr_ref, bl_ref, u_ref, v_ref):
        hb = h_ref[...]
        u_ref[...] = jnp.dot(hb, wl_ref[...], preferred_element_type=jnp.float32,
                             precision=lax.Precision.HIGHEST)
        v_ref[...] = (
            jnp.dot(hb, wr_ref[...], preferred_element_type=jnp.float32) + bl_ref[...]
        )

    return pl.pallas_call(
        body,
        grid=(N // BR,),
        in_specs=[
            pl.BlockSpec((BR, k), lambda i: (i, 0)),
            pl.BlockSpec((k, HID), lambda i: (0, 0)),
            pl.BlockSpec((k, HID), lambda i: (0, 0)),
            pl.BlockSpec((1, HID), lambda i: (0, 0)),
        ],
        out_specs=[
            pl.BlockSpec((BR, HID), lambda i: (i, 0)),
            pl.BlockSpec((BR, HID), lambda i: (i, 0)),
        ],
        out_shape=[jax.ShapeDtypeStruct((N, HID), jnp.float32)] * 2,
    )(h, wlt, wrt, bl)


def _tc_post1(s2, cnt2, v, g, b):
    def body(s_ref, c_ref, v_ref, g_ref, b_ref, o_ref):
        cnt = c_ref[0][:, 0:1]
        f = s_ref[0] / jnp.maximum(cnt, 1.0) + v_ref[...]
        o_ref[...] = _layernorm(_gelu(f), g_ref[...], b_ref[...])

    return pl.pallas_call(
        body,
        grid=(2, NBH),
        in_specs=[
            pl.BlockSpec((1, BR, HID), lambda c, i: (c, i, 0)),
            pl.BlockSpec((1, BR, 16), lambda c, i: (c, i, 0)),
            pl.BlockSpec((BR, HID), lambda c, i: (c * NBH + i, 0)),
            pl.BlockSpec((1, HID), lambda c, i: (0, 0)),
            pl.BlockSpec((1, HID), lambda c, i: (0, 0)),
        ],
        out_specs=pl.BlockSpec((BR, HID), lambda c, i: (c * NBH + i, 0)),
        out_shape=jax.ShapeDtypeStruct((N, HID), jnp.float32),
    )(s2, cnt2, v, g, b)


def _tc_layer(s2, cnt2, h, wlt, wrt, bl, g, b):
    def body(s_ref, c_ref, h_ref, wl_ref, wr_ref, bl_ref, g_ref, b_ref, o_ref):
        cnt = c_ref[0][:, 0:1]
        agg = s_ref[0] / jnp.maximum(cnt, 1.0)
        hb = h_ref[...]
        f = (
            jnp.dot(agg, wl_ref[...], preferred_element_type=jnp.float32)
            + bl_ref[...]
            + jnp.dot(hb, wr_ref[...], preferred_element_type=jnp.float32)
        )
        f = _layernorm(_gelu(f), g_ref[...], b_ref[...])
        o_ref[...] = f + hb

    return pl.pallas_call(
        body,
        grid=(2, NBH),
        in_specs=[
            pl.BlockSpec((1, BR, HID), lambda c, i: (c, i, 0)),
            pl.BlockSpec((1, BR, 16), lambda c, i: (c, i, 0)),
            pl.BlockSpec((BR, HID), lambda c, i: (c * NBH + i, 0)),
            pl.BlockSpec((HID, HID), lambda c, i: (0, 0)),
            pl.BlockSpec((HID, HID), lambda c, i: (0, 0)),
            pl.BlockSpec((1, HID), lambda c, i: (0, 0)),
            pl.BlockSpec((1, HID), lambda c, i: (0, 0)),
            pl.BlockSpec((1, HID), lambda c, i: (0, 0)),
        ],
        out_specs=pl.BlockSpec((BR, HID), lambda c, i: (c * NBH + i, 0)),
        out_shape=jax.ShapeDtypeStruct((N, HID), jnp.float32),
    )(s2, cnt2, h, wlt, wrt, bl, g, b)


def _tc_pool(h, batch3):
    PB = 1000

    def body(h_ref, b_ref, o_ref):
        i = pl.program_id(0)

        @pl.when(i == 0)
        def _():
            o_ref[...] = jnp.zeros_like(o_ref)

        ids = b_ref[0, 0, :]
        oh = (lax.broadcasted_iota(jnp.int32, (G, PB), 0) == ids[None, :]).astype(
            jnp.float32
        )
        o_ref[...] += jnp.dot(oh, h_ref[...], preferred_element_type=jnp.float32,
                              precision=lax.Precision.HIGHEST)

    return pl.pallas_call(
        body,
        grid=(N // PB,),
        in_specs=[
            pl.BlockSpec((PB, HID), lambda i: (i, 0)),
            pl.BlockSpec((1, 1, PB), lambda i: (i, 0, 0)),
        ],
        out_specs=pl.BlockSpec((G, HID), lambda i: (0, 0)),
        out_shape=jax.ShapeDtypeStruct((G, HID), jnp.float32),
    )(h, batch3)


def _tc_mlp(pool, gf, w0a, w0b, b0, g0, lb0, w1, b1, g1, lb1, w2, b2):
    def body(p_ref, gf_ref, w0a_ref, w0b_ref, b0_ref, g0_ref, lb0_ref, w1_ref,
             b1_ref, g1_ref, lb1_ref, w2_ref, b2_ref, o_ref):
        f = (
            jnp.dot(p_ref[...], w0a_ref[...], preferred_element_type=jnp.float32)
            + jnp.dot(gf_ref[...], w0b_ref[...], preferred_element_type=jnp.float32)
            + b0_ref[...]
        )
        f = _layernorm(_gelu(f), g0_ref[...], lb0_ref[...])
        f2 = jnp.dot(f, w1_ref[...], preferred_element_type=jnp.float32) + b1_ref[...]
        f2 = _layernorm(_gelu(f2), g1_ref[...], lb1_ref[...]) + f
        o_ref[...] = jnp.sum(f2 * w2_ref[...], axis=1, keepdims=True) + b2_ref[...]

    return pl.pallas_call(
        body,
        out_shape=jax.ShapeDtypeStruct((G, 1), jnp.float32),
    )(pool, gf, w0a, w0b, b0, g0, lb0, w1, b1, g1, lb1, w2, b2)


def kernel(x, edge_index, batch, global_features, params):
    src = edge_index[0]
    dst = edge_index[1]
    batch3 = batch.reshape(N // 1000, 1, 1000)

    cnt2 = _sc_counts(dst)
    esrc, edstl, ecnt = _sc_partition(src, dst)

    p = params["sage"][0]
    u, v = _tc_pre(x, p["Wl"].T, p["Wr"].T, p["bl"].reshape(1, HID))
    s2 = _sc_segsum2(u, esrc, edstl, ecnt)
    h = _tc_post1(s2, cnt2, v, p["g"].reshape(1, HID), p["b"].reshape(1, HID))

    for p in params["sage"][1:]:
        s2 = _sc_segsum2(h, esrc, edstl, ecnt)
        h = _tc_layer(
            s2, cnt2, h, p["Wl"].T, p["Wr"].T, p["bl"].reshape(1, HID),
            p["g"].reshape(1, HID), p["b"].reshape(1, HID),
        )

    pool = _tc_pool(h, batch3)

    m = params["mlp"]
    w0 = m["W0"].T
    return _tc_mlp(
        pool,
        global_features,
        w0[:HID],
        w0[HID:],
        m["b0"].reshape(1, 32),
        m["g0"].reshape(1, 32),
        m["lb0"].reshape(1, 32),
        m["W1"].T,
        m["b1"].reshape(1, 32),
        m["g1"].reshape(1, 32),
        m["lb1"].reshape(1, 32),
        m["W2"].reshape(1, 32),
        m["b2"].reshape(1, 1),
    )
